# Initial kernel scaffold; baseline (speedup 1.0000x reference)
import jax
import jax.numpy as jnp
from jax.experimental import pallas as pl


def kernel(node_features, edge_features, edge_index, Wn1, bn1, Wn2, bn2, We1, be1, We2, be2):
    raise NotImplementedError("write your pallas kernel here")


# speedup vs baseline: 2.0528x; 2.0528x over previous
import functools

import jax
import jax.numpy as jnp
from jax import lax
from jax.experimental import pallas as pl
from jax.experimental.pallas import tpu as pltpu
from jax.experimental.pallas import tpu_sc as plsc

N = 10000
E = 320000
D_NODE = 128
D_EDGE = 16
H = 64

N_PAD = 10240
EB = 6400
NB = 2048
EB_SC = 3200
E_QUARTER = E // 4
COLS = 8
ACC = COLS * N_PAD
TMP = 8192


def _edge_mlp_kernel(x_ref, w1_ref, b1_ref, w2_ref, b2_ref, out_ref):
    h = lax.dot_general(w1_ref[...], x_ref[...], (((1,), (1,)), ((), ())),
                        preferred_element_type=jnp.float32)
    h = jax.nn.relu(h + b1_ref[...])
    o = lax.dot_general(w2_ref[...], h, (((1,), (0,)), ((), ())),
                        preferred_element_type=jnp.float32)
    out_ref[...] = jax.nn.relu(o + b2_ref[...])


def _edge_mlp_T(edge_features, We1, be1, We2, be2):
    grid = (E // EB,)
    return pl.pallas_call(
        _edge_mlp_kernel,
        grid=grid,
        in_specs=[
            pl.BlockSpec((EB, D_EDGE), lambda i: (i, 0)),
            pl.BlockSpec((H, D_EDGE), lambda i: (0, 0)),
            pl.BlockSpec((H, 1), lambda i: (0, 0)),
            pl.BlockSpec((H, H), lambda i: (0, 0)),
            pl.BlockSpec((H, 1), lambda i: (0, 0)),
        ],
        out_specs=pl.BlockSpec((H, EB), lambda i: (0, i)),
        out_shape=jax.ShapeDtypeStruct((H, E), jnp.float32),
    )(edge_features, We1, be1.reshape(H, 1), We2, be2.reshape(H, 1))


def _sc_agg_body(src_hbm, embT_hbm, out_hbm, src_v, emb_v, acc_v, tmp_v, spmem):
    cid = lax.axis_index("c")
    sid = lax.axis_index("s")
    eg = sid // 8
    cg = sid % 8
    q = cid * 2 + eg
    col0 = cg * COLS

    def zinit(j, c):
        acc_v[pl.ds(j * 16, 16)] = jnp.zeros((16,), jnp.float32)
        return c
    lax.fori_loop(0, ACC // 16, zinit, 0)

    def chunk_body(t, c):
        e0 = q * E_QUARTER + t * EB_SC
        pltpu.sync_copy(src_hbm.at[pl.ds(e0, EB_SC)], src_v)
        pltpu.sync_copy(embT_hbm.at[pl.ds(col0, COLS), pl.ds(e0, EB_SC)], emb_v)

        def vec_body(i, c2):
            idx = src_v[pl.ds(i * 16, 16)]
            occ, _ = plsc.scan_count(idx)

            def do_round(mask):
                for cc in range(COLS):
                    a = idx + (cc * N_PAD)
                    v = emb_v[cc, pl.ds(i * 16, 16)]
                    old = plsc.load_gather(acc_v, [a], mask=mask)
                    plsc.store_scatter(acc_v, [a], jnp.maximum(old, v), mask=mask)

            do_round(occ == 1)
            nmax = jnp.max(occ)

            def wbody(r):
                do_round(occ == r)
                return r + 1

            lax.while_loop(lambda r: r <= nmax, wbody, jnp.int32(2))
            return c2

        lax.fori_loop(0, EB_SC // 16, vec_body, 0)
        return c

    lax.fori_loop(0, E_QUARTER // EB_SC, chunk_body, 0)

    for k in range(ACC // TMP):
        @pl.when(eg == 1)
        def _():
            pltpu.sync_copy(acc_v.at[pl.ds(k * TMP, TMP)], spmem.at[cg])

        plsc.subcore_barrier()

        @pl.when(eg == 0)
        def _():
            pltpu.sync_copy(spmem.at[cg], tmp_v)

            def mbody(j, c):
                o = k * TMP + j * 16
                acc_v[pl.ds(o, 16)] = jnp.maximum(
                    acc_v[pl.ds(o, 16)], tmp_v[pl.ds(j * 16, 16)])
                return c
            lax.fori_loop(0, TMP // 16, mbody, 0)

        plsc.subcore_barrier()

    @pl.when(eg == 0)
    def _():
        pltpu.sync_copy(acc_v, out_hbm.at[cid, pl.ds(col0 * N_PAD, ACC)])


def _sc_agg(src, embT):
    mesh = plsc.VectorSubcoreMesh(core_axis_name="c", subcore_axis_name="s",
                                  num_cores=2, num_subcores=16)
    kfn = pl.kernel(
        _sc_agg_body,
        out_type=jax.ShapeDtypeStruct((2, H * N_PAD), jnp.float32),
        mesh=mesh,
        compiler_params=pltpu.CompilerParams(needs_layout_passes=False),
        scratch_types=[
            pltpu.VMEM((EB_SC,), jnp.int32),
            pltpu.VMEM((COLS, EB_SC), jnp.float32),
            pltpu.VMEM((ACC,), jnp.float32),
            pltpu.VMEM((TMP,), jnp.float32),
            pltpu.VMEM_SHARED((8, TMP), jnp.float32),
        ],
    )
    return kfn(src, embT)


def _node_kernel(x_ref, w1_ref, b1_ref, w2_ref, b2_ref, p0_ref, p1_ref, out_ref):
    h = lax.dot_general(w1_ref[...], x_ref[...], (((1,), (1,)), ((), ())),
                        preferred_element_type=jnp.float32)
    h = jax.nn.relu(h + b1_ref[...])
    o = lax.dot_general(w2_ref[...], h, (((1,), (0,)), ((), ())),
                        preferred_element_type=jnp.float32)
    out_ref[0:H, :] = jax.nn.relu(o + b2_ref[...])
    out_ref[H:2 * H, :] = jnp.maximum(p0_ref[...], p1_ref[...])


def _node_mlp_concat_T(node_features_pad, Wn1, bn1, Wn2, bn2, part0, part1):
    grid = (N_PAD // NB,)
    return pl.pallas_call(
        _node_kernel,
        grid=grid,
        in_specs=[
            pl.BlockSpec((NB, D_NODE), lambda i: (i, 0)),
            pl.BlockSpec((H, D_NODE), lambda i: (0, 0)),
            pl.BlockSpec((H, 1), lambda i: (0, 0)),
            pl.BlockSpec((H, H), lambda i: (0, 0)),
            pl.BlockSpec((H, 1), lambda i: (0, 0)),
            pl.BlockSpec((H, NB), lambda i: (0, i)),
            pl.BlockSpec((H, NB), lambda i: (0, i)),
        ],
        out_specs=pl.BlockSpec((2 * H, NB), lambda i: (0, i)),
        out_shape=jax.ShapeDtypeStruct((2 * H, N_PAD), jnp.float32),
    )(node_features_pad, Wn1, bn1.reshape(H, 1), Wn2, bn2.reshape(H, 1),
      part0, part1)


@jax.jit
def kernel(node_features, edge_features, edge_index, Wn1, bn1, Wn2, bn2,
           We1, be1, We2, be2):
    src = edge_index[0]
    embT = _edge_mlp_T(edge_features, We1, be1, We2, be2)
    parts = _sc_agg(src, embT)
    part0 = parts[0].reshape(H, N_PAD)
    part1 = parts[1].reshape(H, N_PAD)
    x_pad = jnp.pad(node_features, ((0, N_PAD - N), (0, 0)))
    outT = _node_mlp_concat_T(x_pad, Wn1, bn1, Wn2, bn2, part0, part1)
    return outT[:, :N].T


# speedup vs baseline: 2.3338x; 1.1369x over previous
import functools

import jax
import jax.numpy as jnp
from jax import lax
from jax.experimental import pallas as pl
from jax.experimental.pallas import tpu as pltpu
from jax.experimental.pallas import tpu_sc as plsc

N = 10000
E = 320000
D_NODE = 128
D_EDGE = 16
H = 64

N_PAD = 10240
EB = 6400
NB = 2048
EB_SC = 640
E_QUARTER = E // 4
NCHUNK = E_QUARTER // EB_SC
COLS = 8
ACC = COLS * N
TMP = 16000


def _edge_mlp_kernel(x_ref, w1_ref, b1_ref, w2_ref, b2_ref, out_ref):
    h = lax.dot_general(w1_ref[...], x_ref[...], (((1,), (1,)), ((), ())),
                        preferred_element_type=jnp.float32)
    h = jax.nn.relu(h + b1_ref[...])
    o = lax.dot_general(w2_ref[...], h, (((1,), (0,)), ((), ())),
                        preferred_element_type=jnp.float32)
    out_ref[...] = jax.nn.relu(o + b2_ref[...])


def _edge_mlp_T(edge_features, We1, be1, We2, be2):
    grid = (E // EB,)
    return pl.pallas_call(
        _edge_mlp_kernel,
        grid=grid,
        in_specs=[
            pl.BlockSpec((EB, D_EDGE), lambda i: (i, 0)),
            pl.BlockSpec((H, D_EDGE), lambda i: (0, 0)),
            pl.BlockSpec((H, 1), lambda i: (0, 0)),
            pl.BlockSpec((H, H), lambda i: (0, 0)),
            pl.BlockSpec((H, 1), lambda i: (0, 0)),
        ],
        out_specs=pl.BlockSpec((H, EB), lambda i: (0, i)),
        out_shape=jax.ShapeDtypeStruct((H, E), jnp.float32),
    )(edge_features, We1, be1.reshape(H, 1), We2, be2.reshape(H, 1))


def _sc_agg_body(src_hbm, embT_hbm, out_hbm, src_v, emb_v, acc_v, tmp_v,
                 sem_s, sem_e, spmem):
    cid = lax.axis_index("c")
    sid = lax.axis_index("s")
    eg = sid // 8
    cg = sid % 8
    q = cid * 2 + eg
    col0 = cg * COLS
    ebase = q * E_QUARTER

    def zinit(j, c):
        acc_v[pl.ds(j * 16, 16)] = jnp.zeros((16,), jnp.float32)
        return c
    lax.fori_loop(0, ACC // 16, zinit, 0)

    def dma_pair(t, slot):
        e0 = ebase + t * EB_SC
        return (
            pltpu.make_async_copy(src_hbm.at[pl.ds(e0, EB_SC)],
                                  src_v.at[slot], sem_s.at[slot]),
            pltpu.make_async_copy(
                embT_hbm.at[pl.ds(col0, COLS), pl.ds(e0, EB_SC)],
                emb_v.at[slot], sem_e.at[slot]),
        )

    def start_chunk(t, slot):
        a, b = dma_pair(t, slot)
        a.start()
        b.start()

    def wait_chunk(t, slot):
        a, b = dma_pair(t, slot)
        a.wait()
        b.wait()

    def process(slot):
        def vec_body(i, c2):
            def one(i):
                idx = src_v[slot, pl.ds(i * 16, 16)]
                occ, _ = plsc.scan_count(idx)

                def do_round(mask):
                    for cc in range(COLS):
                        a = idx + (cc * N)
                        v = emb_v[slot, cc, pl.ds(i * 16, 16)]
                        old = plsc.load_gather(acc_v, [a], mask=mask)
                        plsc.store_scatter(acc_v, [a], jnp.maximum(old, v),
                                           mask=mask)

                do_round(occ == 1)
                nmax = jnp.max(occ)

                def wbody(r):
                    do_round(occ == r)
                    return r + 1

                lax.while_loop(lambda r: r <= nmax, wbody, jnp.int32(2))

            one(2 * i)
            one(2 * i + 1)
            return c2

        lax.fori_loop(0, EB_SC // 32, vec_body, 0)

    start_chunk(0, 0)

    def chunk_pair(tt, c):
        t = 2 * tt
        start_chunk(t + 1, 1)
        wait_chunk(t, 0)
        process(0)
        start_chunk(t + 2, 0)
        wait_chunk(t + 1, 1)
        process(1)
        return c

    lax.fori_loop(0, (NCHUNK - 1) // 2, chunk_pair, 0)
    wait_chunk(NCHUNK - 1, 0)
    process(0)

    for k in range(ACC // TMP):
        @pl.when(eg == 1)
        def _():
            pltpu.sync_copy(acc_v.at[pl.ds(k * TMP, TMP)], spmem.at[cg])

        plsc.subcore_barrier()

        @pl.when(eg == 0)
        def _():
            pltpu.sync_copy(spmem.at[cg], tmp_v)

            def mbody(j, c):
                o = k * TMP + j * 16
                acc_v[pl.ds(o, 16)] = jnp.maximum(
                    acc_v[pl.ds(o, 16)], tmp_v[pl.ds(j * 16, 16)])
                return c
            lax.fori_loop(0, TMP // 16, mbody, 0)

        plsc.subcore_barrier()

    @pl.when(eg == 0)
    def _():
        for cc in range(COLS):
            pltpu.sync_copy(acc_v.at[pl.ds(cc * N, N)],
                            out_hbm.at[pl.ds(cid * (H * N_PAD) + (col0 + cc) * N_PAD, N)])


def _sc_agg(src, embT):
    mesh = plsc.VectorSubcoreMesh(core_axis_name="c", subcore_axis_name="s",
                                  num_cores=2, num_subcores=16)
    kfn = pl.kernel(
        _sc_agg_body,
        out_type=jax.ShapeDtypeStruct((2 * H * N_PAD,), jnp.float32),
        mesh=mesh,
        compiler_params=pltpu.CompilerParams(needs_layout_passes=False),
        scratch_types=[
            pltpu.VMEM((2, EB_SC), jnp.int32),
            pltpu.VMEM((2, COLS, EB_SC), jnp.float32),
            pltpu.VMEM((ACC,), jnp.float32),
            pltpu.VMEM((TMP,), jnp.float32),
            pltpu.SemaphoreType.DMA((2,)),
            pltpu.SemaphoreType.DMA((2,)),
            pltpu.VMEM_SHARED((8, TMP), jnp.float32),
        ],
    )
    return kfn(src, embT)


def _node_kernel(x_ref, w1_ref, b1_ref, w2_ref, b2_ref, p0_ref, p1_ref, out_ref):
    h = lax.dot_general(w1_ref[...], x_ref[...], (((1,), (1,)), ((), ())),
                        preferred_element_type=jnp.float32)
    h = jax.nn.relu(h + b1_ref[...])
    o = lax.dot_general(w2_ref[...], h, (((1,), (0,)), ((), ())),
                        preferred_element_type=jnp.float32)
    out_ref[0:H, :] = jax.nn.relu(o + b2_ref[...])
    out_ref[H:2 * H, :] = jnp.maximum(p0_ref[...], p1_ref[...])


def _node_mlp_concat_T(node_features_pad, Wn1, bn1, Wn2, bn2, part0, part1):
    grid = (N_PAD // NB,)
    return pl.pallas_call(
        _node_kernel,
        grid=grid,
        in_specs=[
            pl.BlockSpec((NB, D_NODE), lambda i: (i, 0)),
            pl.BlockSpec((H, D_NODE), lambda i: (0, 0)),
            pl.BlockSpec((H, 1), lambda i: (0, 0)),
            pl.BlockSpec((H, H), lambda i: (0, 0)),
            pl.BlockSpec((H, 1), lambda i: (0, 0)),
            pl.BlockSpec((H, NB), lambda i: (0, i)),
            pl.BlockSpec((H, NB), lambda i: (0, i)),
        ],
        out_specs=pl.BlockSpec((2 * H, NB), lambda i: (0, i)),
        out_shape=jax.ShapeDtypeStruct((2 * H, N_PAD), jnp.float32),
    )(node_features_pad, Wn1, bn1.reshape(H, 1), Wn2, bn2.reshape(H, 1),
      part0, part1)


@jax.jit
def kernel(node_features, edge_features, edge_index, Wn1, bn1, Wn2, bn2,
           We1, be1, We2, be2):
    src = edge_index[0]
    embT = _edge_mlp_T(edge_features, We1, be1, We2, be2)
    parts = _sc_agg(src, embT).reshape(2, H, N_PAD)
    part0 = parts[0]
    part1 = parts[1]
    x_pad = jnp.pad(node_features, ((0, N_PAD - N), (0, 0)))
    outT = _node_mlp_concat_T(x_pad, Wn1, bn1, Wn2, bn2, part0, part1)
    return outT[:, :N].T
